# Initial kernel scaffold; baseline (speedup 1.0000x reference)
import jax
import jax.numpy as jnp
from jax.experimental import pallas as pl


def kernel(x, emb_E, emb_R1, gamma, beta):
    raise NotImplementedError("write your pallas kernel here")


# speedup vs baseline: 2.2900x; 2.2900x over previous
import functools

import jax
import jax.numpy as jnp
from jax import lax
from jax.experimental import pallas as pl
from jax.experimental.pallas import tpu as pltpu
from jax.experimental.pallas import tpu_sc as plsc

BATCH = 16384
DIM = 128
EPS = 1e-5

NC = 2
NS = 16
NW = NC * NS
BPW = BATCH // NW
IDX_ROWS = BPW // 128

_MESH = plsc.VectorSubcoreMesh(core_axis_name="c", subcore_axis_name="s")


@functools.partial(
    pl.kernel,
    out_type=[
        jax.ShapeDtypeStruct((BATCH, DIM), jnp.float32),
        jax.ShapeDtypeStruct((BATCH, DIM), jnp.float32),
        jax.ShapeDtypeStruct((BATCH, DIM), jnp.float32),
    ],
    mesh=_MESH,
    scratch_types=[
        pltpu.VMEM((IDX_ROWS, 128), jnp.int32),
        pltpu.VMEM((BPW, DIM), jnp.float32),
        pltpu.SemaphoreType.DMA,
    ],
)
def _sc_gather(s_hbm, r_hbm, o_hbm, emb_e, emb_r, es_out, er_out, eo_out,
               idx_v, rows_v, sem):
    wid = lax.axis_index("s") * NC + lax.axis_index("c")
    base = wid * BPW

    def one_gather(idx_hbm, table, out_hbm):
        pltpu.sync_copy(idx_hbm.at[wid], idx_v)
        copies = []
        for j in range(IDX_ROWS):
            copies.append(pltpu.async_copy(
                table.at[idx_v.at[j]],
                rows_v.at[pl.ds(j * 128, 128)], sem))
        for c in copies:
            c.wait()
        pltpu.sync_copy(rows_v, out_hbm.at[pl.ds(base, BPW)])

    one_gather(s_hbm, emb_e, es_out)
    one_gather(r_hbm, emb_r, er_out)
    one_gather(o_hbm, emb_e, eo_out)


def _bn_body(es_ref, g_ref, b_ref, out_ref):
    es = es_ref[...]
    mean = jnp.mean(es, axis=0, keepdims=True)
    var = jnp.mean((es - mean) ** 2, axis=0, keepdims=True)
    out_ref[...] = (es - mean) / jnp.sqrt(var + EPS) * g_ref[...] + b_ref[...]


_bn = pl.pallas_call(
    _bn_body,
    out_shape=jax.ShapeDtypeStruct((BATCH, DIM), jnp.float32),
)


def kernel(x, emb_E, emb_R1, gamma, beta):
    s = x[:, 0].reshape(NW, IDX_ROWS, 128)
    r = x[:, 1].reshape(NW, IDX_ROWS, 128)
    o = x[:, 2].reshape(NW, IDX_ROWS, 128)
    es_raw, er, eo = _sc_gather(s, r, o, emb_E, emb_R1)
    es = _bn(es_raw, gamma.reshape(1, DIM), beta.reshape(1, DIM))
    return (es, er, eo)
